# baseline (speedup 1.0000x reference)
import functools

import jax
import jax.numpy as jnp
from jax import lax
from jax.experimental import pallas as pl
from jax.experimental.pallas import tpu as pltpu
from jax.experimental.pallas import tpu_sc as plsc

_L = 128
_G = 16
_NW = 32
_NBUF = 4
_SUBG = 2


def _gather_blocks(ids_t, tab_pairs):
    s_len, b_len = ids_t.shape
    _, d2 = tab_pairs.shape
    d = d2 // 2
    ng = _L // _G

    mesh = plsc.VectorSubcoreMesh(core_axis_name="c", subcore_axis_name="s")

    @functools.partial(
        pl.kernel,
        out_type=jax.ShapeDtypeStruct((s_len, d, b_len), jnp.float32),
        mesh=mesh,
        compiler_params=pltpu.CompilerParams(
            use_tc_tiling_on_sc=True, needs_layout_passes=False),
        scratch_types=[
            pltpu.VMEM((s_len, _L), jnp.int32),
            pltpu.VMEM((_NBUF, _L), jnp.int32),
            pltpu.VMEM((_NBUF, _L), jnp.int32),
            pltpu.VMEM((_NBUF, _L, d2), jnp.float32),
            pltpu.VMEM((2, d, _L), jnp.float32),
            pltpu.SemaphoreType.DMA,
            pltpu.SemaphoreType.DMA,
            pltpu.SemaphoreType.DMA,
            pltpu.SemaphoreType.DMA,
            pltpu.SemaphoreType.DMA,
            pltpu.SemaphoreType.DMA,
            pltpu.SemaphoreType.DMA,
        ],
    )
    def grab(ids_hbm, tab_hbm, out_hbm, idx_v, pidx_v, poff_v, pair_v,
             out_v, si, sg0, sg1, sg2, sg3, so0, so1):
        gat_sems = [sg0, sg1, sg2, sg3]
        out_sems = [so0, so1]

        nc = lax.axis_size("c")
        wid = lax.axis_index("s") * nc + lax.axis_index("c")
        col0 = wid * _L

        def gather_copies(slot):
            n_sub = _L // _SUBG
            return [
                pltpu.make_async_copy(
                    tab_hbm.at[pidx_v.at[slot, pl.ds(i * n_sub, n_sub)]],
                    pair_v.at[slot, pl.ds(i * n_sub, n_sub)],
                    gat_sems[slot])
                for i in range(_SUBG)
            ]

        def out_copy(s, slot):
            return pltpu.make_async_copy(
                out_v.at[slot], out_hbm.at[s, :, pl.ds(col0, _L)],
                out_sems[slot])

        def prep(s, slot):
            for g in range(ng):
                v = idx_v[s, pl.ds(g * _G, _G)]
                pidx_v[slot, pl.ds(g * _G, _G)] = lax.shift_right_logical(
                    v, 1)
                poff_v[slot, pl.ds(g * _G, _G)] = lax.shift_left(
                    lax.bitwise_and(v, 1), 6)

        def transpose(gslot, oslot):
            rows = [lax.iota(jnp.int32, _G) + g * _G for g in range(ng)]
            cols0 = tuple(poff_v[gslot, pl.ds(g * _G, _G)]
                          for g in range(ng))

            @pl.loop(0, d, init_carry=cols0, unroll=2)
            def _(k, cols):
                vecs = [plsc.load_gather(pair_v.at[gslot],
                                         [rows[g], cols[g]])
                        for g in range(ng)]
                for g in range(ng):
                    out_v[oslot, k, pl.ds(g * _G, _G)] = vecs[g]
                return tuple(c + 1 for c in cols)

        def step(s, gslot, oslot, head, tail):
            if not tail:
                nslot = (gslot + 3) % _NBUF
                prep(s + 3, nslot)
                for c in gather_copies(nslot):
                    c.start()
            for c in gather_copies(gslot):
                c.wait()
            if not head:
                out_copy(s - 2, oslot).wait()
            transpose(gslot, oslot)
            out_copy(s, oslot).start()

        pltpu.make_async_copy(
            ids_hbm.at[:, pl.ds(col0, _L)], idx_v, si).start()
        pltpu.make_async_copy(
            ids_hbm.at[:, pl.ds(col0, _L)], idx_v, si).wait()
        for j in range(_NBUF - 1):
            prep(j, j)
            for c in gather_copies(j):
                c.start()

        step(0, 0, 0, head=True, tail=False)
        step(1, 1, 1, head=True, tail=False)
        step(2, 2, 0, head=False, tail=False)
        step(3, 3, 1, head=False, tail=False)

        @pl.loop(4, s_len - 4, step=_NBUF)
        def _(s):
            step(s, 0, 0, head=False, tail=False)
            step(s + 1, 1, 1, head=False, tail=False)
            step(s + 2, 2, 0, head=False, tail=False)
            step(s + 3, 3, 1, head=False, tail=False)

        step(s_len - 4, 0, 0, head=False, tail=False)
        step(s_len - 3, 1, 1, head=False, tail=True)
        step(s_len - 2, 2, 0, head=False, tail=True)
        step(s_len - 1, 3, 1, head=False, tail=True)
        out_copy(s_len - 2, 0).wait()
        out_copy(s_len - 1, 1).wait()

    return grab(ids_t, tab_pairs)


def kernel(input_ids, word_embeddings):
    b, s = input_ids.shape
    v, d = word_embeddings.shape
    ids_t = input_ids.T.astype(jnp.int32)
    tab_pairs = word_embeddings.reshape(v // 2, 2 * d)
    out_t = _gather_blocks(ids_t, tab_pairs)
    return out_t.transpose(2, 0, 1)


# speedup vs baseline: 1.5368x; 1.5368x over previous
import functools

import jax
import jax.numpy as jnp
from jax import lax
from jax.experimental import pallas as pl
from jax.experimental.pallas import tpu as pltpu
from jax.experimental.pallas import tpu_sc as plsc

_L = 128
_G = 16
_NW = 32
_NBUF = 4
_SUBG = 2


def _gather_blocks(ids_t, tab_pairs):
    s_len, b_len = ids_t.shape
    _, d2 = tab_pairs.shape
    d = d2 // 2
    ng = _L // _G

    mesh = plsc.VectorSubcoreMesh(core_axis_name="c", subcore_axis_name="s")

    @functools.partial(
        pl.kernel,
        out_type=jax.ShapeDtypeStruct((s_len, d, b_len), jnp.float32),
        mesh=mesh,
        compiler_params=pltpu.CompilerParams(
            use_tc_tiling_on_sc=True, needs_layout_passes=False),
        scratch_types=[
            pltpu.VMEM((s_len, _L), jnp.int32),
            pltpu.VMEM((_NBUF, _L), jnp.int32),
            pltpu.VMEM((_NBUF, _L), jnp.int32),
            pltpu.VMEM((_NBUF, _L, d2), jnp.float32),
            pltpu.VMEM((2, d, _L), jnp.float32),
            pltpu.SemaphoreType.DMA,
            pltpu.SemaphoreType.DMA,
            pltpu.SemaphoreType.DMA,
            pltpu.SemaphoreType.DMA,
            pltpu.SemaphoreType.DMA,
            pltpu.SemaphoreType.DMA,
            pltpu.SemaphoreType.DMA,
        ],
    )
    def grab(ids_hbm, tab_hbm, out_hbm, idx_v, pidx_v, poff_v, pair_v,
             out_v, si, sg0, sg1, sg2, sg3, so0, so1):
        gat_sems = [sg0, sg1, sg2, sg3]
        out_sems = [so0, so1]

        nc = lax.axis_size("c")
        wid = lax.axis_index("s") * nc + lax.axis_index("c")
        col0 = wid * _L

        def gather_copies(slot):
            n_sub = _L // _SUBG
            return [
                pltpu.make_async_copy(
                    tab_hbm.at[pidx_v.at[slot, pl.ds(i * n_sub, n_sub)]],
                    pair_v.at[slot, pl.ds(i * n_sub, n_sub)],
                    gat_sems[slot])
                for i in range(_SUBG)
            ]

        def out_copy(s, slot):
            return pltpu.make_async_copy(
                out_v.at[slot], out_hbm.at[s, :, pl.ds(col0, _L)],
                out_sems[slot])

        def prep(s, slot):
            for g in range(ng):
                v = idx_v[s, pl.ds(g * _G, _G)]
                pidx_v[slot, pl.ds(g * _G, _G)] = lax.shift_right_logical(
                    v, 1)
                poff_v[slot, pl.ds(g * _G, _G)] = lax.shift_left(
                    lax.bitwise_and(v, 1), 6)

        def transpose(gslot, oslot):
            iot = lax.iota(jnp.int32, _G)
            rows = [iot + g * _G for g in range(ng)]
            poffs = [poff_v[gslot, pl.ds(g * _G, _G)] for g in range(ng)]
            kv0 = tuple([iot] * ng)

            @pl.loop(0, d, init_carry=kv0, unroll=4)
            def _(k0, kvs):
                vecs = [plsc.load_gather(pair_v.at[gslot],
                                         [rows[g], kvs[g] + poffs[g]])
                        for g in range(ng)]
                for g in range(ng):
                    plsc.store_scatter(out_v.at[oslot],
                                       [kvs[g], rows[g]], vecs[g])
                return tuple(
                    lax.bitwise_and(kv + 1, d - 1) for kv in kvs)

        def step(s, gslot, oslot, head, tail):
            if not tail:
                nslot = (gslot + 3) % _NBUF
                prep(s + 3, nslot)
                for c in gather_copies(nslot):
                    c.start()
            for c in gather_copies(gslot):
                c.wait()
            if not head:
                out_copy(s - 2, oslot).wait()
            transpose(gslot, oslot)
            out_copy(s, oslot).start()

        pltpu.make_async_copy(
            ids_hbm.at[:, pl.ds(col0, _L)], idx_v, si).start()
        pltpu.make_async_copy(
            ids_hbm.at[:, pl.ds(col0, _L)], idx_v, si).wait()
        for j in range(_NBUF - 1):
            prep(j, j)
            for c in gather_copies(j):
                c.start()

        step(0, 0, 0, head=True, tail=False)
        step(1, 1, 1, head=True, tail=False)
        step(2, 2, 0, head=False, tail=False)
        step(3, 3, 1, head=False, tail=False)

        @pl.loop(4, s_len - 4, step=_NBUF)
        def _(s):
            step(s, 0, 0, head=False, tail=False)
            step(s + 1, 1, 1, head=False, tail=False)
            step(s + 2, 2, 0, head=False, tail=False)
            step(s + 3, 3, 1, head=False, tail=False)

        step(s_len - 4, 0, 0, head=False, tail=False)
        step(s_len - 3, 1, 1, head=False, tail=True)
        step(s_len - 2, 2, 0, head=False, tail=True)
        step(s_len - 1, 3, 1, head=False, tail=True)
        out_copy(s_len - 2, 0).wait()
        out_copy(s_len - 1, 1).wait()

    return grab(ids_t, tab_pairs)


def kernel(input_ids, word_embeddings):
    b, s = input_ids.shape
    v, d = word_embeddings.shape
    ids_t = input_ids.T.astype(jnp.int32)
    tab_pairs = word_embeddings.reshape(v // 2, 2 * d)
    out_t = _gather_blocks(ids_t, tab_pairs)
    return out_t.transpose(2, 0, 1)


# speedup vs baseline: 2.1025x; 1.3681x over previous
import functools

import jax
import jax.numpy as jnp
from jax import lax
from jax.experimental import pallas as pl
from jax.experimental.pallas import tpu as pltpu
from jax.experimental.pallas import tpu_sc as plsc

_L = 128
_G = 16
_NW = 32
_NBUF = 4
_SUBG = 2


def _gather_blocks(ids_t, tab_pad):
    s_len, b_len = ids_t.shape
    _, d2 = tab_pad.shape
    d = d2 // 2
    ng = _L // _G

    mesh = plsc.VectorSubcoreMesh(core_axis_name="c", subcore_axis_name="s")

    @functools.partial(
        pl.kernel,
        out_type=jax.ShapeDtypeStruct((s_len, d, b_len), jnp.float32),
        mesh=mesh,
        compiler_params=pltpu.CompilerParams(
            use_tc_tiling_on_sc=True, needs_layout_passes=False),
        scratch_types=[
            pltpu.VMEM((s_len, _L), jnp.int32),
            pltpu.VMEM((_NBUF, _L, d2), jnp.float32),
            pltpu.VMEM((2, d, _L), jnp.float32),
            pltpu.SemaphoreType.DMA,
            pltpu.SemaphoreType.DMA,
            pltpu.SemaphoreType.DMA,
            pltpu.SemaphoreType.DMA,
            pltpu.SemaphoreType.DMA,
            pltpu.SemaphoreType.DMA,
            pltpu.SemaphoreType.DMA,
        ],
    )
    def grab(ids_hbm, tab_hbm, out_hbm, idx_v, row_v, out_v,
             si, sg0, sg1, sg2, sg3, so0, so1):
        gat_sems = [sg0, sg1, sg2, sg3]
        out_sems = [so0, so1]

        nc = lax.axis_size("c")
        wid = lax.axis_index("s") * nc + lax.axis_index("c")
        col0 = wid * _L

        def gather_copies(s, slot):
            n_sub = _L // _SUBG
            return [
                pltpu.make_async_copy(
                    tab_hbm.at[idx_v.at[s, pl.ds(i * n_sub, n_sub)]],
                    row_v.at[slot, pl.ds(i * n_sub, n_sub)],
                    gat_sems[slot])
                for i in range(_SUBG)
            ]

        def out_copy(s, slot):
            return pltpu.make_async_copy(
                out_v.at[slot], out_hbm.at[s, :, pl.ds(col0, _L)],
                out_sems[slot])

        def transpose(gslot, oslot):
            iot = lax.iota(jnp.int32, _G)
            rows = [iot + g * _G for g in range(ng)]

            @pl.loop(0, d, init_carry=iot, unroll=4)
            def _(k0, kv):
                vecs = [plsc.load_gather(row_v.at[gslot], [rows[g], kv])
                        for g in range(ng)]
                for g in range(ng):
                    plsc.store_scatter(out_v.at[oslot],
                                       [kv, rows[g]], vecs[g])
                return lax.bitwise_and(kv + 1, d - 1)

        def step(s, gslot, oslot, head, tail):
            if not tail:
                for c in gather_copies(s + 3, (gslot + 3) % _NBUF):
                    c.start()
            for c in gather_copies(s, gslot):
                c.wait()
            if not head:
                out_copy(s - 2, oslot).wait()
            transpose(gslot, oslot)
            out_copy(s, oslot).start()

        pltpu.make_async_copy(
            ids_hbm.at[:, pl.ds(col0, _L)], idx_v, si).start()
        pltpu.make_async_copy(
            ids_hbm.at[:, pl.ds(col0, _L)], idx_v, si).wait()
        for j in range(_NBUF - 1):
            for c in gather_copies(j, j):
                c.start()

        step(0, 0, 0, head=True, tail=False)
        step(1, 1, 1, head=True, tail=False)
        step(2, 2, 0, head=False, tail=False)
        step(3, 3, 1, head=False, tail=False)

        @pl.loop(4, s_len - 4, step=_NBUF)
        def _(s):
            step(s, 0, 0, head=False, tail=False)
            step(s + 1, 1, 1, head=False, tail=False)
            step(s + 2, 2, 0, head=False, tail=False)
            step(s + 3, 3, 1, head=False, tail=False)

        step(s_len - 4, 0, 0, head=False, tail=False)
        step(s_len - 3, 1, 1, head=False, tail=True)
        step(s_len - 2, 2, 0, head=False, tail=True)
        step(s_len - 1, 3, 1, head=False, tail=True)
        out_copy(s_len - 2, 0).wait()
        out_copy(s_len - 1, 1).wait()

    return grab(ids_t, tab_pad)


def kernel(input_ids, word_embeddings):
    b, s = input_ids.shape
    v, d = word_embeddings.shape
    ids_t = input_ids.T.astype(jnp.int32)
    tab_pad = jnp.pad(word_embeddings.astype(jnp.float32), ((0, 0), (0, d)))
    out_t = _gather_blocks(ids_t, tab_pad)
    return out_t.transpose(2, 0, 1)


# speedup vs baseline: 3.0421x; 1.4469x over previous
import functools

import jax
import jax.numpy as jnp
from jax import lax
from jax.experimental import pallas as pl
from jax.experimental.pallas import tpu as pltpu
from jax.experimental.pallas import tpu_sc as plsc

_L = 128
_G = 16
_NW = 32
_NBUF = 4
_SUBG = 2


def _gather_blocks(ids_t, tab_pad):
    s_len, b_len = ids_t.shape
    _, d2 = tab_pad.shape
    d = d2 // 2
    ng = _L // _G

    mesh = plsc.VectorSubcoreMesh(core_axis_name="c", subcore_axis_name="s")

    @functools.partial(
        pl.kernel,
        out_type=jax.ShapeDtypeStruct((s_len, d, b_len), jnp.float32),
        mesh=mesh,
        compiler_params=pltpu.CompilerParams(
            use_tc_tiling_on_sc=True, needs_layout_passes=False),
        scratch_types=[
            pltpu.VMEM((s_len, _L), jnp.int32),
            pltpu.VMEM((_NBUF, _L, d2), jnp.float32),
            pltpu.VMEM((2, d, _L), jnp.float32),
            pltpu.SemaphoreType.DMA,
            pltpu.SemaphoreType.DMA,
            pltpu.SemaphoreType.DMA,
            pltpu.SemaphoreType.DMA,
            pltpu.SemaphoreType.DMA,
            pltpu.SemaphoreType.DMA,
            pltpu.SemaphoreType.DMA,
        ],
    )
    def grab(ids_hbm, tab_hbm, out_hbm, idx_v, row_v, out_v,
             si, sg0, sg1, sg2, sg3, so0, so1):
        gat_sems = [sg0, sg1, sg2, sg3]
        out_sems = [so0, so1]

        nc = lax.axis_size("c")
        wid = lax.axis_index("s") * nc + lax.axis_index("c")
        col0 = wid * _L

        def gather_copies(s, slot):
            n_sub = _L // _SUBG
            return [
                pltpu.make_async_copy(
                    tab_hbm.at[idx_v.at[s, pl.ds(i * n_sub, n_sub)]],
                    row_v.at[slot, pl.ds(i * n_sub, n_sub)],
                    gat_sems[slot])
                for i in range(_SUBG)
            ]

        def out_copy(s, slot):
            return pltpu.make_async_copy(
                out_v.at[slot], out_hbm.at[s, :, pl.ds(col0, _L)],
                out_sems[slot])

        def transpose(gslot, oslot):
            iot = lax.iota(jnp.int32, _G)
            rows = [iot + g * _G for g in range(ng)]

            @pl.loop(0, d, init_carry=iot, unroll=4)
            def _(k0, kv):
                vecs = [plsc.load_gather(row_v.at[gslot], [rows[g], kv])
                        for g in range(ng)]
                for g in range(ng):
                    plsc.store_scatter(out_v.at[oslot],
                                       [kv, rows[g]], vecs[g])
                return lax.bitwise_and(kv + 1, d - 1)

        def step(s, gslot, oslot, head, tail):
            if not tail:
                for c in gather_copies(s + 3, (gslot + 3) % _NBUF):
                    c.start()
            for c in gather_copies(s, gslot):
                c.wait()
            if not head:
                out_copy(s - 2, oslot).wait()
            transpose(gslot, oslot)
            out_copy(s, oslot).start()

        pltpu.make_async_copy(
            ids_hbm.at[:, pl.ds(col0, _L)], idx_v, si).start()
        pltpu.make_async_copy(
            ids_hbm.at[:, pl.ds(col0, _L)], idx_v, si).wait()
        for j in range(_NBUF - 1):
            for c in gather_copies(j, j):
                c.start()

        step(0, 0, 0, head=True, tail=False)
        step(1, 1, 1, head=True, tail=False)
        step(2, 2, 0, head=False, tail=False)
        step(3, 3, 1, head=False, tail=False)

        @pl.loop(4, s_len - 4, step=_NBUF)
        def _(s):
            step(s, 0, 0, head=False, tail=False)
            step(s + 1, 1, 1, head=False, tail=False)
            step(s + 2, 2, 0, head=False, tail=False)
            step(s + 3, 3, 1, head=False, tail=False)

        step(s_len - 4, 0, 0, head=False, tail=False)
        step(s_len - 3, 1, 1, head=False, tail=True)
        step(s_len - 2, 2, 0, head=False, tail=True)
        step(s_len - 1, 3, 1, head=False, tail=True)
        out_copy(s_len - 2, 0).wait()
        out_copy(s_len - 1, 1).wait()

    return grab(ids_t, tab_pad)


def _detile_pad(tab_t, tail_pad):
    d, v = tab_t.shape
    d2 = 2 * d
    nb_full = v // _L
    rem = v - nb_full * _L
    per_w = nb_full // _NW
    nb_extra = nb_full - per_w * _NW
    ng = _L // _G

    mesh = plsc.VectorSubcoreMesh(core_axis_name="c", subcore_axis_name="s")

    @functools.partial(
        pl.kernel,
        out_type=jax.ShapeDtypeStruct((v, d2), jnp.float32),
        mesh=mesh,
        compiler_params=pltpu.CompilerParams(
            use_tc_tiling_on_sc=True, needs_layout_passes=False),
        scratch_types=[
            pltpu.VMEM((2, d, _L), jnp.float32),
            pltpu.VMEM((2, _L, d2), jnp.float32),
            pltpu.SemaphoreType.DMA,
            pltpu.SemaphoreType.DMA,
            pltpu.SemaphoreType.DMA,
            pltpu.SemaphoreType.DMA,
        ],
    )
    def conv(tab_hbm, tail_hbm, out_hbm, in_v, tout, si0, si1, so0, so1):
        in_sems = [si0, si1]
        out_sems = [so0, so1]

        nc = lax.axis_size("c")
        wid = lax.axis_index("s") * nc + lax.axis_index("c")

        def in_copy(c, slot):
            return pltpu.make_async_copy(
                tab_hbm.at[:, pl.ds(c * _L, _L)], in_v.at[slot],
                in_sems[slot])

        def out_copy(c, slot):
            return pltpu.make_async_copy(
                tout.at[slot], out_hbm.at[pl.ds(c * _L, _L), :],
                out_sems[slot])

        def trans(slot, ngroups):
            iot = lax.iota(jnp.int32, _G)
            rows = [iot + g * _G for g in range(ngroups)]

            @pl.loop(0, d, init_carry=iot, unroll=4)
            def _(d0, kv):
                vecs = [plsc.load_gather(in_v.at[slot], [kv, rows[g]])
                        for g in range(ngroups)]
                for g in range(ngroups):
                    plsc.store_scatter(tout.at[slot],
                                       [rows[g], kv], vecs[g])
                return lax.bitwise_and(kv + 1, d - 1)

        def step(m, slot, head, tail):
            c = wid + _NW * m
            in_copy(c, slot).wait()
            if not head:
                out_copy(0, slot).wait()
            trans(slot, ng)
            out_copy(c, slot).start()
            if not tail:
                in_copy(wid + _NW * (m + 2), slot).start()

        in_copy(wid, 0).start()
        in_copy(wid + _NW, 1).start()
        step(0, 0, head=True, tail=False)
        step(1, 1, head=True, tail=False)

        @pl.loop(2, per_w - 2, step=2)
        def _(m):
            step(m, 0, head=False, tail=False)
            step(m + 1, 1, head=False, tail=False)

        step(per_w - 2, 0, head=False, tail=True)
        step(per_w - 1, 1, head=False, tail=True)
        out_copy(0, 0).wait()
        out_copy(0, 1).wait()

        @pl.when(wid < nb_extra)
        def _():
            c = per_w * _NW + wid
            in_copy(c, 0).start()
            in_copy(c, 0).wait()
            trans(0, ng)
            out_copy(c, 0).start()
            out_copy(c, 0).wait()

        @pl.when(wid == nb_extra)
        def _():
            pltpu.make_async_copy(tail_hbm, in_v.at[0], si0).start()
            pltpu.make_async_copy(tail_hbm, in_v.at[0], si0).wait()
            pltpu.make_async_copy(
                in_v.at[0, pl.ds(0, rem)],
                out_hbm.at[pl.ds(nb_full * _L, rem), :], so0).start()
            pltpu.make_async_copy(
                in_v.at[0, pl.ds(0, rem)],
                out_hbm.at[pl.ds(nb_full * _L, rem), :], so0).wait()

    return conv(tab_t, tail_pad)


def kernel(input_ids, word_embeddings):
    b, s = input_ids.shape
    v, d = word_embeddings.shape
    ids_t = input_ids.T.astype(jnp.int32)
    we = word_embeddings.astype(jnp.float32)
    rem = v % _L
    tail_pad = jnp.pad(we[v - rem:], ((0, 0), (0, d)))
    tab_pad = _detile_pad(we.T, tail_pad)
    out_t = _gather_blocks(ids_t, tab_pad)
    return out_t.transpose(2, 0, 1)


# speedup vs baseline: 3.3805x; 1.1112x over previous
import functools

import jax
import jax.numpy as jnp
from jax import lax
from jax.experimental import pallas as pl
from jax.experimental.pallas import tpu as pltpu
from jax.experimental.pallas import tpu_sc as plsc

_L = 128
_G = 16
_NW = 32
_NBUF = 4
_SUBG = 2


def _gather_blocks(ids_t, tab_pad):
    s_len, b_len = ids_t.shape
    _, d2 = tab_pad.shape
    d = d2 // 2
    ng = _L // _G

    mesh = plsc.VectorSubcoreMesh(core_axis_name="c", subcore_axis_name="s")

    @functools.partial(
        pl.kernel,
        out_type=jax.ShapeDtypeStruct((s_len, d, b_len), jnp.float32),
        mesh=mesh,
        compiler_params=pltpu.CompilerParams(
            use_tc_tiling_on_sc=True, needs_layout_passes=False),
        scratch_types=[
            pltpu.VMEM((s_len, _L), jnp.int32),
            pltpu.VMEM((_NBUF, _L), jnp.int32),
            pltpu.VMEM((_NBUF, _L), jnp.int32),
            pltpu.VMEM((_NBUF, _L, d2), jnp.float32),
            pltpu.VMEM((2, d, _L), jnp.float32),
            pltpu.SemaphoreType.DMA,
            pltpu.SemaphoreType.DMA,
            pltpu.SemaphoreType.DMA,
            pltpu.SemaphoreType.DMA,
            pltpu.SemaphoreType.DMA,
            pltpu.SemaphoreType.DMA,
            pltpu.SemaphoreType.DMA,
        ],
    )
    def grab(ids_hbm, tab_hbm, out_hbm, idx_v, pidx_v, poff_v, row_v,
             out_v, si, sg0, sg1, sg2, sg3, so0, so1):
        gat_sems = [sg0, sg1, sg2, sg3]
        out_sems = [so0, so1]

        nc = lax.axis_size("c")
        wid = lax.axis_index("s") * nc + lax.axis_index("c")
        col0 = wid * _L

        def gather_copies(slot):
            n_sub = _L // _SUBG
            return [
                pltpu.make_async_copy(
                    tab_hbm.at[pidx_v.at[slot, pl.ds(i * n_sub, n_sub)]],
                    row_v.at[slot, pl.ds(i * n_sub, n_sub)],
                    gat_sems[slot])
                for i in range(_SUBG)
            ]

        def prep(s, slot):
            for g in range(ng):
                v = idx_v[s, pl.ds(g * _G, _G)]
                pidx_v[slot, pl.ds(g * _G, _G)] = lax.shift_right_logical(
                    v, 1)
                poff_v[slot, pl.ds(g * _G, _G)] = lax.shift_left(
                    lax.bitwise_and(v, 1), 6)

        def out_copy(s, slot):
            return pltpu.make_async_copy(
                out_v.at[slot], out_hbm.at[s, :, pl.ds(col0, _L)],
                out_sems[slot])

        def transpose(gslot, oslot):
            iot = lax.iota(jnp.int32, _G)
            rows = [iot + g * _G for g in range(ng)]
            poffs = [poff_v[gslot, pl.ds(g * _G, _G)] for g in range(ng)]

            @pl.loop(0, d, init_carry=iot, unroll=4)
            def _(k0, kv):
                vecs = [plsc.load_gather(row_v.at[gslot],
                                         [rows[g], kv + poffs[g]])
                        for g in range(ng)]
                for g in range(ng):
                    plsc.store_scatter(out_v.at[oslot],
                                       [kv, rows[g]], vecs[g])
                return lax.bitwise_and(kv + 1, d - 1)

        def step(s, gslot, oslot, head, tail):
            if not tail:
                nslot = (gslot + 3) % _NBUF
                prep(s + 3, nslot)
                for c in gather_copies(nslot):
                    c.start()
            for c in gather_copies(gslot):
                c.wait()
            if not head:
                out_copy(s - 2, oslot).wait()
            transpose(gslot, oslot)
            out_copy(s, oslot).start()

        pltpu.make_async_copy(
            ids_hbm.at[:, pl.ds(col0, _L)], idx_v, si).start()
        pltpu.make_async_copy(
            ids_hbm.at[:, pl.ds(col0, _L)], idx_v, si).wait()
        for j in range(_NBUF - 1):
            prep(j, j)
            for c in gather_copies(j):
                c.start()

        step(0, 0, 0, head=True, tail=False)
        step(1, 1, 1, head=True, tail=False)
        step(2, 2, 0, head=False, tail=False)
        step(3, 3, 1, head=False, tail=False)

        @pl.loop(4, s_len - 4, step=_NBUF)
        def _(s):
            step(s, 0, 0, head=False, tail=False)
            step(s + 1, 1, 1, head=False, tail=False)
            step(s + 2, 2, 0, head=False, tail=False)
            step(s + 3, 3, 1, head=False, tail=False)

        step(s_len - 4, 0, 0, head=False, tail=False)
        step(s_len - 3, 1, 1, head=False, tail=True)
        step(s_len - 2, 2, 0, head=False, tail=True)
        step(s_len - 1, 3, 1, head=False, tail=True)
        out_copy(s_len - 2, 0).wait()
        out_copy(s_len - 1, 1).wait()

    return grab(ids_t, tab_pad)


def _detile_pad(tab_t, tail_pad):
    d, v = tab_t.shape
    d2 = 2 * d
    nb_full = v // _L
    rem = v - nb_full * _L
    per_w = nb_full // _NW
    nb_extra = nb_full - per_w * _NW
    ng = _L // _G

    mesh = plsc.VectorSubcoreMesh(core_axis_name="c", subcore_axis_name="s")

    @functools.partial(
        pl.kernel,
        out_type=jax.ShapeDtypeStruct((v // 2, d2), jnp.float32),
        mesh=mesh,
        compiler_params=pltpu.CompilerParams(
            use_tc_tiling_on_sc=True, needs_layout_passes=False),
        scratch_types=[
            pltpu.VMEM((2, d, _L), jnp.float32),
            pltpu.VMEM((2, _L // 2, d2), jnp.float32),
            pltpu.SemaphoreType.DMA,
            pltpu.SemaphoreType.DMA,
            pltpu.SemaphoreType.DMA,
            pltpu.SemaphoreType.DMA,
        ],
    )
    def conv(tab_hbm, tail_hbm, out_hbm, in_v, tout, si0, si1, so0, so1):
        in_sems = [si0, si1]
        out_sems = [so0, so1]

        nc = lax.axis_size("c")
        wid = lax.axis_index("s") * nc + lax.axis_index("c")

        def in_copy(c, slot):
            return pltpu.make_async_copy(
                tab_hbm.at[:, pl.ds(c * _L, _L)], in_v.at[slot],
                in_sems[slot])

        def out_copy(c, slot):
            return pltpu.make_async_copy(
                tout.at[slot], out_hbm.at[pl.ds(c * (_L // 2), _L // 2), :],
                out_sems[slot])

        def trans(slot, ngroups):
            iot = lax.iota(jnp.int32, _G)
            rows = [iot + g * _G for g in range(ngroups)]
            qrows = [lax.shift_right_logical(r, 1) for r in rows]
            coffs = [lax.shift_left(lax.bitwise_and(r, 1), 6) for r in rows]

            @pl.loop(0, d, init_carry=iot, unroll=4)
            def _(d0, kv):
                vecs = [plsc.load_gather(in_v.at[slot], [kv, rows[g]])
                        for g in range(ngroups)]
                for g in range(ngroups):
                    plsc.store_scatter(tout.at[slot],
                                       [qrows[g], kv + coffs[g]], vecs[g])
                return lax.bitwise_and(kv + 1, d - 1)

        def step(m, slot, head, tail):
            c = wid + _NW * m
            in_copy(c, slot).wait()
            if not head:
                out_copy(0, slot).wait()
            trans(slot, ng)
            out_copy(c, slot).start()
            if not tail:
                in_copy(wid + _NW * (m + 2), slot).start()

        in_copy(wid, 0).start()
        in_copy(wid + _NW, 1).start()
        step(0, 0, head=True, tail=False)
        step(1, 1, head=True, tail=False)

        @pl.loop(2, per_w - 2, step=2)
        def _(m):
            step(m, 0, head=False, tail=False)
            step(m + 1, 1, head=False, tail=False)

        step(per_w - 2, 0, head=False, tail=True)
        step(per_w - 1, 1, head=False, tail=True)
        out_copy(0, 0).wait()
        out_copy(0, 1).wait()

        @pl.when(wid < nb_extra)
        def _():
            c = per_w * _NW + wid
            in_copy(c, 0).start()
            in_copy(c, 0).wait()
            trans(0, ng)
            out_copy(c, 0).start()
            out_copy(c, 0).wait()

        @pl.when(wid == nb_extra)
        def _():
            nq = rem // 2
            pltpu.make_async_copy(
                tail_hbm, in_v.at[0, pl.ds(0, nq)], si0).start()
            pltpu.make_async_copy(
                tail_hbm, in_v.at[0, pl.ds(0, nq)], si0).wait()
            pltpu.make_async_copy(
                in_v.at[0, pl.ds(0, nq)],
                out_hbm.at[pl.ds(nb_full * (_L // 2), nq), :], so0).start()
            pltpu.make_async_copy(
                in_v.at[0, pl.ds(0, nq)],
                out_hbm.at[pl.ds(nb_full * (_L // 2), nq), :], so0).wait()

    return conv(tab_t, tail_pad)


def kernel(input_ids, word_embeddings):
    b, s = input_ids.shape
    v, d = word_embeddings.shape
    ids_t = input_ids.T.astype(jnp.int32)
    we = word_embeddings.astype(jnp.float32)
    rem = v % _L
    tail_pairs = we[v - rem:].reshape(rem // 2, 2 * d)
    tab_pairs = _detile_pad(we.T, tail_pairs)
    out_t = _gather_blocks(ids_t, tab_pairs)
    return out_t.transpose(2, 0, 1)
